# Initial kernel scaffold; baseline (speedup 1.0000x reference)
import jax
import jax.numpy as jnp
from jax.experimental import pallas as pl


def kernel(x, edge_index_supplies, edge_index_competes, Wl0_s, bl0_s, Wr0_s, Wl0_c, bl0_c, Wr0_c, Wl1_s, bl1_s, Wr1_s, Wl1_c, bl1_c, Wr1_c, W_out, b_out):
    raise NotImplementedError("write your pallas kernel here")


# speedup vs baseline: 5.0058x; 5.0058x over previous
import functools

import jax
import jax.numpy as jnp
from jax import lax
from jax.experimental import pallas as pl
from jax.experimental.pallas import tpu as pltpu
from jax.experimental.pallas import tpu_sc as plsc

N = 10000
E = 320000
D = 128
D_OUT = 64

NC = 2
NS = 16
NW = NC * NS

CH = 128
N_CHUNKS = E // CH
CHUNKS_PER_TILE = -(-N_CHUNKS // NW)

N_PAD = 10240
ROWS_PER_TILE = N_PAD // NS

BN = 1000
GRID = N // BN


def _seg_body(compute_cnt, x_hbm, src_hbm, dst_hbm, *rest):
  if compute_cnt:
    (out_agg, out_cnt, acc, cnt, zbuf, cbuf, ones,
     idx_s, idx_d, rows, sem) = rest
  else:
    out_agg, acc, zbuf, idx_s, idx_d, rows, sem = rest

  cid = lax.axis_index("c")
  sid = lax.axis_index("s")
  wid = sid * NC + cid

  def zloop(i, _):
    zbuf[i // 8, pl.ds((i % 8) * 16, 16)] = jnp.zeros((16,), jnp.float32)
    return 0
  lax.fori_loop(0, 128 * (D // 16), zloop, 0)
  row0 = sid * ROWS_PER_TILE
  for t in range(ROWS_PER_TILE // 128):
    pltpu.sync_copy(zbuf, acc.at[pl.ds(row0 + t * 128, 128)])

  if compute_cnt:
    def czloop(i, _):
      cbuf[pl.ds(i * 16, 16)] = jnp.zeros((16,), jnp.float32)
      return 0
    lax.fori_loop(0, ROWS_PER_TILE // 16, czloop, 0)
    pltpu.sync_copy(cbuf, cnt.at[pl.ds(row0, ROWS_PER_TILE)])
    def oloop(i, _):
      ones[pl.ds(i * 16, 16)] = jnp.ones((16,), jnp.float32)
      return 0
    lax.fori_loop(0, CH // 16, oloop, 0)

  plsc.subcore_barrier()

  def chunk(j, _):
    r = j * NW + wid
    @pl.when(r < N_CHUNKS)
    def _():
      pltpu.sync_copy(src_hbm.at[r], idx_s.at[0])
      pltpu.sync_copy(dst_hbm.at[r], idx_d.at[0])
      pltpu.async_copy(x_hbm.at[idx_s.at[0]], rows, sem).wait()
      pltpu.sync_copy(rows, acc.at[idx_d.at[0]], add=True)
      if compute_cnt:
        pltpu.sync_copy(ones, cnt.at[idx_d.at[0]], add=True)
    return 0
  lax.fori_loop(0, CHUNKS_PER_TILE, chunk, 0)

  plsc.subcore_barrier()

  for t in range(ROWS_PER_TILE // 128):
    pltpu.sync_copy(acc.at[pl.ds(row0 + t * 128, 128)], rows)
    pltpu.sync_copy(rows, out_agg.at[cid, pl.ds(row0 + t * 128, 128)])
  if compute_cnt:
    pltpu.sync_copy(cnt.at[pl.ds(row0, ROWS_PER_TILE)], cbuf)
    pltpu.sync_copy(cbuf, out_cnt.at[cid, pl.ds(row0, ROWS_PER_TILE)])


def _make_segsum(compute_cnt):
  mesh = plsc.VectorSubcoreMesh(core_axis_name="c", subcore_axis_name="s",
                                num_cores=NC, num_subcores=NS)
  out_type = [jax.ShapeDtypeStruct((NC, N_PAD, D), jnp.float32)]
  scratch = [
      pltpu.VMEM_SHARED((N_PAD, D), jnp.float32),
  ]
  if compute_cnt:
    out_type.append(jax.ShapeDtypeStruct((NC, N_PAD), jnp.float32))
    scratch.append(pltpu.VMEM_SHARED((N_PAD,), jnp.float32))
  scratch.append(pltpu.VMEM((128, D), jnp.float32))
  if compute_cnt:
    scratch.append(pltpu.VMEM((ROWS_PER_TILE,), jnp.float32))
    scratch.append(pltpu.VMEM((CH,), jnp.float32))
  scratch += [
      pltpu.VMEM((1, CH), jnp.int32),
      pltpu.VMEM((1, CH), jnp.int32),
      pltpu.VMEM((CH, D), jnp.float32),
      pltpu.SemaphoreType.DMA,
  ]
  return pl.kernel(
      functools.partial(_seg_body, compute_cnt),
      out_type=tuple(out_type),
      mesh=mesh,
      scratch_types=tuple(scratch),
  )


_segsum_cnt = _make_segsum(True)
_segsum = _make_segsum(False)


def _tc_body(last, aggs_ref, aggc_ref, cnts_ref, cntc_ref, x_ref,
             wls_ref, wlc_ref, wrs_ref, wrc_ref, bs_ref, bc_ref,
             *rest):
  if last:
    wout_ref, bout_ref, out_ref = rest
  else:
    (out_ref,) = rest
  aggs = aggs_ref[0] + aggs_ref[1]
  aggc = aggc_ref[0] + aggc_ref[1]
  cs = cnts_ref[0, 0] + cnts_ref[0, 1]
  cc = cntc_ref[0, 0] + cntc_ref[0, 1]
  means = aggs * (1.0 / jnp.maximum(cs, 1.0))[:, None]
  meanc = aggc * (1.0 / jnp.maximum(cc, 1.0))[:, None]
  x = x_ref[...]
  h = (jnp.dot(means, wls_ref[...], preferred_element_type=jnp.float32)
       + jnp.dot(meanc, wlc_ref[...], preferred_element_type=jnp.float32)
       + jnp.dot(x, wrs_ref[...] + wrc_ref[...],
                 preferred_element_type=jnp.float32)
       + bs_ref[...] + bc_ref[...])
  h = jnp.maximum(h, 0.0)
  if last:
    out_ref[...] = (jnp.dot(h, wout_ref[...], preferred_element_type=jnp.float32)
                    + bout_ref[...])
  else:
    out_ref[...] = h


def _make_tc(last):
  agg_spec = pl.BlockSpec((NC, BN, D), lambda i: (0, i, 0))
  cnt_spec = pl.BlockSpec((1, NC, BN), lambda i: (i, 0, 0))
  x_spec = pl.BlockSpec((BN, D), lambda i: (i, 0))
  w_spec = pl.BlockSpec((D, D), lambda i: (0, 0))
  b_spec = pl.BlockSpec((1, D), lambda i: (0, 0))
  in_specs = [agg_spec, agg_spec, cnt_spec, cnt_spec, x_spec,
              w_spec, w_spec, w_spec, w_spec, b_spec, b_spec]
  if last:
    in_specs += [pl.BlockSpec((D, D_OUT), lambda i: (0, 0)),
                 pl.BlockSpec((1, D_OUT), lambda i: (0, 0))]
    out_spec = pl.BlockSpec((BN, D_OUT), lambda i: (i, 0))
    out_shape = jax.ShapeDtypeStruct((N, D_OUT), jnp.float32)
  else:
    out_spec = pl.BlockSpec((BN, D), lambda i: (i, 0))
    out_shape = jax.ShapeDtypeStruct((N, D), jnp.float32)
  return pl.pallas_call(
      functools.partial(_tc_body, last),
      grid=(GRID,),
      in_specs=in_specs,
      out_specs=out_spec,
      out_shape=out_shape,
  )


_tc_layer = _make_tc(False)
_tc_last = _make_tc(True)


def kernel(x, edge_index_supplies, edge_index_competes,
           Wl0_s, bl0_s, Wr0_s, Wl0_c, bl0_c, Wr0_c,
           Wl1_s, bl1_s, Wr1_s, Wl1_c, bl1_c, Wr1_c,
           W_out, b_out):
  src_s = edge_index_supplies[0].reshape(N_CHUNKS, CH)
  dst_s = edge_index_supplies[1].reshape(N_CHUNKS, CH)
  src_c = edge_index_competes[0].reshape(N_CHUNKS, CH)
  dst_c = edge_index_competes[1].reshape(N_CHUNKS, CH)

  agg_s0, cnt_s = _segsum_cnt(x, src_s, dst_s)
  agg_c0, cnt_c = _segsum_cnt(x, src_c, dst_c)

  cnt_s_r = cnt_s[:, :N].reshape(NC, GRID, BN).transpose(1, 0, 2)
  cnt_c_r = cnt_c[:, :N].reshape(NC, GRID, BN).transpose(1, 0, 2)

  bl0_s2 = bl0_s.reshape(1, D)
  bl0_c2 = bl0_c.reshape(1, D)
  bl1_s2 = bl1_s.reshape(1, D)
  bl1_c2 = bl1_c.reshape(1, D)
  b_out2 = b_out.reshape(1, D_OUT)

  h = _tc_layer(agg_s0, agg_c0, cnt_s_r, cnt_c_r, x,
                Wl0_s, Wl0_c, Wr0_s, Wr0_c, bl0_s2, bl0_c2)

  (agg_s1,) = _segsum(h, src_s, dst_s)
  (agg_c1,) = _segsum(h, src_c, dst_c)

  out = _tc_last(agg_s1, agg_c1, cnt_s_r, cnt_c_r, h,
                 Wl1_s, Wl1_c, Wr1_s, Wr1_c, bl1_s2, bl1_c2,
                 W_out, b_out2)
  return out


# speedup vs baseline: 9.3163x; 1.8611x over previous
import functools

import jax
import jax.numpy as jnp
from jax import lax
from jax.experimental import pallas as pl
from jax.experimental.pallas import tpu as pltpu
from jax.experimental.pallas import tpu_sc as plsc

N = 10000
E = 320000
D = 128
D_OUT = 64

NC = 2
NS = 16

CH = 125
N_CHUNKS = E // CH
CPT = N_CHUNKS // NS
NB = 2
GROUPS = CPT // NB

N_PAD = 10240
ROWS_PER_TILE = N_PAD // NS

BN = 1000
GRID = N // BN


def _seg_body(compute_cnt, x_hbm, src_hbm, dst_hbm, *rest):
  if compute_cnt:
    (out_agg, out_cnt, acc, cnt, zbuf, cbuf, ones, idx_sv, idx_dv) = rest[:9]
    rest = rest[9:]
  else:
    (out_agg, acc, zbuf, idx_sv, idx_dv) = rest[:5]
    rest = rest[5:]
  rows, rest = rest[:NB], rest[NB:]
  gsems, rest = rest[:NB], rest[NB:]
  ssems, rest = rest[:NB], rest[NB:]
  if compute_cnt:
    csems, rest = rest[:NB], rest[NB:]
  isems = rest[:2]

  cid = lax.axis_index("c")
  sid = lax.axis_index("s")

  def zloop(i, _):
    zbuf[i // 8, pl.ds((i % 8) * 16, 16)] = jnp.zeros((16,), jnp.float32)
    return 0
  lax.fori_loop(0, 32 * (D // 16), zloop, 0)
  row0 = sid * ROWS_PER_TILE
  for t in range(ROWS_PER_TILE // 32):
    pltpu.sync_copy(zbuf, acc.at[pl.ds(row0 + t * 32, 32)])

  if compute_cnt:
    def czloop(i, _):
      cbuf[pl.ds(i * 16, 16)] = jnp.zeros((16,), jnp.float32)
      return 0
    lax.fori_loop(0, ROWS_PER_TILE // 16, czloop, 0)
    pltpu.sync_copy(cbuf, cnt.at[pl.ds(row0, ROWS_PER_TILE)])
    def oloop(i, _):
      ones[pl.ds(i * 16, 16)] = jnp.ones((16,), jnp.float32)
      return 0
    lax.fori_loop(0, 8, oloop, 0)

  base = sid * CPT
  pltpu.sync_copy(src_hbm.at[cid, pl.ds(base, NB)], idx_sv.at[0])
  pltpu.sync_copy(dst_hbm.at[cid, pl.ds(base, NB)], idx_dv.at[0])
  pltpu.async_copy(x_hbm.at[idx_sv.at[0, 0]], rows[0], gsems[0])

  plsc.subcore_barrier()

  def group(g, _):
    p = g % 3
    q = (g + 1) % 3
    @pl.when(g + 1 < GROUPS)
    def _():
      nxt = base + (g + 1) * NB
      pltpu.async_copy(src_hbm.at[cid, pl.ds(nxt, NB)], idx_sv.at[q],
                       isems[0])
      pltpu.async_copy(dst_hbm.at[cid, pl.ds(nxt, NB)], idx_dv.at[q],
                       isems[1])
    for b in range(NB):
      o = 1 - b
      pltpu.make_async_copy(x_hbm.at[idx_sv.at[p, b]], rows[b],
                            gsems[b]).wait()
      pltpu.async_copy(rows[b], acc.at[idx_dv.at[p, b]], ssems[b], add=True)
      if compute_cnt:
        pltpu.async_copy(ones.at[pl.ds(0, CH)], cnt.at[idx_dv.at[p, b]],
                         csems[b], add=True)
      @pl.when(g + b > 0)
      def _(b=b, o=o):
        pltpu.make_async_copy(rows[o], acc.at[idx_dv.at[0, 0]],
                              ssems[o]).wait()
        if compute_cnt:
          pltpu.make_async_copy(ones.at[pl.ds(0, CH)], cnt.at[idx_dv.at[0, 0]],
                                csems[o]).wait()
      if b < NB - 1:
        pltpu.async_copy(x_hbm.at[idx_sv.at[p, b + 1]], rows[o], gsems[o])
      else:
        @pl.when(g + 1 < GROUPS)
        def _(o=o, q=q):
          pltpu.make_async_copy(src_hbm.at[cid, pl.ds(base, NB)],
                                idx_sv.at[q], isems[0]).wait()
          pltpu.make_async_copy(dst_hbm.at[cid, pl.ds(base, NB)],
                                idx_dv.at[q], isems[1]).wait()
          pltpu.async_copy(x_hbm.at[idx_sv.at[q, 0]], rows[o], gsems[o])
    return 0
  lax.fori_loop(0, GROUPS, group, 0)

  last = (CPT - 1) % NB
  pltpu.make_async_copy(rows[last], acc.at[idx_dv.at[0, 0]],
                        ssems[last]).wait()
  if compute_cnt:
    pltpu.make_async_copy(ones.at[pl.ds(0, CH)], cnt.at[idx_dv.at[0, 0]],
                          csems[last]).wait()

  plsc.subcore_barrier()

  pltpu.sync_copy(acc.at[pl.ds(row0, ROWS_PER_TILE)],
                  out_agg.at[cid, pl.ds(row0, ROWS_PER_TILE)])
  if compute_cnt:
    pltpu.sync_copy(cnt.at[pl.ds(row0, ROWS_PER_TILE)],
                    out_cnt.at[cid, pl.ds(row0, ROWS_PER_TILE)])


def _make_segsum(compute_cnt):
  mesh = plsc.VectorSubcoreMesh(core_axis_name="c", subcore_axis_name="s",
                                num_cores=NC, num_subcores=NS)
  out_type = [jax.ShapeDtypeStruct((NC, N_PAD, D), jnp.float32)]
  scratch = [pltpu.VMEM_SHARED((N_PAD, D), jnp.float32)]
  if compute_cnt:
    out_type.append(jax.ShapeDtypeStruct((NC, N_PAD), jnp.float32))
    scratch.append(pltpu.VMEM_SHARED((N_PAD,), jnp.float32))
  scratch.append(pltpu.VMEM((32, D), jnp.float32))
  if compute_cnt:
    scratch.append(pltpu.VMEM((ROWS_PER_TILE,), jnp.float32))
    scratch.append(pltpu.VMEM((128,), jnp.float32))
  scratch += [
      pltpu.VMEM((3, NB, CH), jnp.int32),
      pltpu.VMEM((3, NB, CH), jnp.int32),
  ]
  scratch += [pltpu.VMEM((CH, D), jnp.float32)] * NB
  scratch += [pltpu.SemaphoreType.DMA] * (NB * (3 if compute_cnt else 2) + 2)
  return pl.kernel(
      functools.partial(_seg_body, compute_cnt),
      out_type=tuple(out_type),
      mesh=mesh,
      scratch_types=tuple(scratch),
  )


_segsum_cnt = _make_segsum(True)
_segsum = _make_segsum(False)


def _tc_body(last, agg_ref, cnts_ref, cntc_ref, x_ref,
             wls_ref, wlc_ref, wrs_ref, wrc_ref, bs_ref, bc_ref,
             *rest):
  if last:
    wout_ref, bout_ref, out_ref = rest
  else:
    (out_ref,) = rest
  cs = cnts_ref[0, 0]
  cc = cntc_ref[0, 0]
  means = agg_ref[0] * (1.0 / jnp.maximum(cs, 1.0))[:, None]
  meanc = agg_ref[1] * (1.0 / jnp.maximum(cc, 1.0))[:, None]
  x = x_ref[...]
  h = (jnp.dot(means, wls_ref[...], preferred_element_type=jnp.float32)
       + jnp.dot(meanc, wlc_ref[...], preferred_element_type=jnp.float32)
       + jnp.dot(x, wrs_ref[...] + wrc_ref[...],
                 preferred_element_type=jnp.float32)
       + bs_ref[...] + bc_ref[...])
  h = jnp.maximum(h, 0.0)
  if last:
    out_ref[...] = (jnp.dot(h, wout_ref[...], preferred_element_type=jnp.float32)
                    + bout_ref[...])
  else:
    out_ref[...] = h


def _make_tc(last):
  agg_spec = pl.BlockSpec((NC, BN, D), lambda i: (0, i, 0))
  cnt_spec = pl.BlockSpec((1, 1, BN), lambda i: (i, 0, 0))
  x_spec = pl.BlockSpec((BN, D), lambda i: (i, 0))
  w_spec = pl.BlockSpec((D, D), lambda i: (0, 0))
  b_spec = pl.BlockSpec((1, D), lambda i: (0, 0))
  in_specs = [agg_spec, cnt_spec, cnt_spec, x_spec,
              w_spec, w_spec, w_spec, w_spec, b_spec, b_spec]
  if last:
    in_specs += [pl.BlockSpec((D, D_OUT), lambda i: (0, 0)),
                 pl.BlockSpec((1, D_OUT), lambda i: (0, 0))]
    out_spec = pl.BlockSpec((BN, D_OUT), lambda i: (i, 0))
    out_shape = jax.ShapeDtypeStruct((N, D_OUT), jnp.float32)
  else:
    out_spec = pl.BlockSpec((BN, D), lambda i: (i, 0))
    out_shape = jax.ShapeDtypeStruct((N, D), jnp.float32)
  return pl.pallas_call(
      functools.partial(_tc_body, last),
      grid=(GRID,),
      in_specs=in_specs,
      out_specs=out_spec,
      out_shape=out_shape,
  )


_tc_layer = _make_tc(False)
_tc_last = _make_tc(True)


def kernel(x, edge_index_supplies, edge_index_competes,
           Wl0_s, bl0_s, Wr0_s, Wl0_c, bl0_c, Wr0_c,
           Wl1_s, bl1_s, Wr1_s, Wl1_c, bl1_c, Wr1_c,
           W_out, b_out):
  src_all = jnp.stack([edge_index_supplies[0].reshape(N_CHUNKS, CH),
                       edge_index_competes[0].reshape(N_CHUNKS, CH)])
  dst_all = jnp.stack([edge_index_supplies[1].reshape(N_CHUNKS, CH),
                       edge_index_competes[1].reshape(N_CHUNKS, CH)])

  agg0, cnt_all = _segsum_cnt(x, src_all, dst_all)
  cnt_s_r = cnt_all[0, :N].reshape(GRID, 1, BN)
  cnt_c_r = cnt_all[1, :N].reshape(GRID, 1, BN)

  bl0_s2 = bl0_s.reshape(1, D)
  bl0_c2 = bl0_c.reshape(1, D)
  bl1_s2 = bl1_s.reshape(1, D)
  bl1_c2 = bl1_c.reshape(1, D)
  b_out2 = b_out.reshape(1, D_OUT)

  h = _tc_layer(agg0, cnt_s_r, cnt_c_r, x,
                Wl0_s, Wl0_c, Wr0_s, Wr0_c, bl0_s2, bl0_c2)

  (agg1,) = _segsum(h, src_all, dst_all)

  out = _tc_last(agg1, cnt_s_r, cnt_c_r, h,
                 Wl1_s, Wl1_c, Wr1_s, Wr1_c, bl1_s2, bl1_c2,
                 W_out, b_out2)
  return out


# speedup vs baseline: 11.6922x; 1.2550x over previous
import functools

import jax
import jax.numpy as jnp
from jax import lax
from jax.experimental import pallas as pl
from jax.experimental.pallas import tpu as pltpu
from jax.experimental.pallas import tpu_sc as plsc

N = 10000
E = 320000
D = 128
D_OUT = 64

NC = 2
NS = 16

CH = 100
N_CHUNKS = E // CH
CPT = N_CHUNKS // NS
NB = 2
GROUPS = CPT // NB
RB = 3
SS = (CPT - 2) // 6


N_PAD = 10240
ROWS_PER_TILE = N_PAD // NS

BN = 1000
GRID = N // BN


def _seg_body(compute_cnt, x_hbm, src_hbm, dst_hbm, *rest):
  if compute_cnt:
    (out_agg, out_cnt, acc, cnt, zbuf, cbuf, ones, idx_sv, idx_dv) = rest[:9]
    rest = rest[9:]
  else:
    (out_agg, acc, zbuf, idx_sv, idx_dv) = rest[:5]
    rest = rest[5:]
  rows, rest = rest[:RB], rest[RB:]
  gsems, rest = rest[:RB], rest[RB:]
  ssems, rest = rest[:RB], rest[RB:]
  if compute_cnt:
    csems, rest = rest[:RB], rest[RB:]
  isems = rest[:2]

  cid = lax.axis_index("c")
  sid = lax.axis_index("s")

  def zloop(i, _):
    zbuf[i // 8, pl.ds((i % 8) * 16, 16)] = jnp.zeros((16,), jnp.float32)
    return 0
  lax.fori_loop(0, 32 * (D // 16), zloop, 0)
  row0 = sid * ROWS_PER_TILE
  for t in range(ROWS_PER_TILE // 32):
    pltpu.sync_copy(zbuf, acc.at[pl.ds(row0 + t * 32, 32)])

  if compute_cnt:
    def czloop(i, _):
      cbuf[pl.ds(i * 16, 16)] = jnp.zeros((16,), jnp.float32)
      return 0
    lax.fori_loop(0, ROWS_PER_TILE // 16, czloop, 0)
    pltpu.sync_copy(cbuf, cnt.at[pl.ds(row0, ROWS_PER_TILE)])
    def oloop(i, _):
      ones[pl.ds(i * 16, 16)] = jnp.ones((16,), jnp.float32)
      return 0
    lax.fori_loop(0, 8, oloop, 0)

  base = sid * CPT
  pltpu.sync_copy(src_hbm.at[cid, pl.ds(base, NB)], idx_sv.at[0])
  pltpu.sync_copy(dst_hbm.at[cid, pl.ds(base, NB)], idx_dv.at[0])
  pltpu.sync_copy(src_hbm.at[cid, pl.ds(base + NB, NB)], idx_sv.at[1])
  pltpu.sync_copy(dst_hbm.at[cid, pl.ds(base + NB, NB)], idx_dv.at[1])
  pltpu.async_copy(x_hbm.at[idx_sv.at[0, 0]], rows[0], gsems[0])
  pltpu.async_copy(x_hbm.at[idx_sv.at[0, 1]], rows[1], gsems[1])

  plsc.subcore_barrier()

  def _wait_scatter(r):
    pltpu.make_async_copy(rows[r], acc.at[idx_dv.at[0, 0]], ssems[r]).wait()
    if compute_cnt:
      pltpu.make_async_copy(ones.at[pl.ds(0, CH)], cnt.at[idx_dv.at[0, 0]],
                            csems[r]).wait()

  def _chunk(p, b, r):
    pltpu.make_async_copy(x_hbm.at[idx_sv.at[p, b]], rows[r],
                          gsems[r]).wait()
    pltpu.async_copy(rows[r], acc.at[idx_dv.at[p, b]], ssems[r], add=True)
    if compute_cnt:
      pltpu.async_copy(ones.at[pl.ds(0, CH)], cnt.at[idx_dv.at[p, b]],
                       csems[r], add=True)

  def six(s, _):
    k0 = s * 6
    for kk in range(6):
      b = kk % 2
      p = kk // 2
      r = kk % 3
      rprev = (kk + 2) % 3
      _chunk(p, b, r)
      if kk == 0:
        @pl.when(s >= 1)
        def _():
          _wait_scatter(rprev)
      else:
        _wait_scatter(rprev)
      if b == 0:
        if kk == 0:
          @pl.when(s >= 1)
          def _():
            pltpu.make_async_copy(src_hbm.at[cid, pl.ds(base, NB)],
                                  idx_sv.at[(p + 1) % 3], isems[0]).wait()
            pltpu.make_async_copy(dst_hbm.at[cid, pl.ds(base, NB)],
                                  idx_dv.at[(p + 1) % 3], isems[1]).wait()
        else:
          pltpu.make_async_copy(src_hbm.at[cid, pl.ds(base, NB)],
                                idx_sv.at[(p + 1) % 3], isems[0]).wait()
          pltpu.make_async_copy(dst_hbm.at[cid, pl.ds(base, NB)],
                                idx_dv.at[(p + 1) % 3], isems[1]).wait()
        g = s * 3 + p
        def _prefetch(g=g, p=p):
          nxt = base + (g + 2) * NB
          pltpu.async_copy(src_hbm.at[cid, pl.ds(nxt, NB)],
                           idx_sv.at[(p + 2) % 3], isems[0])
          pltpu.async_copy(dst_hbm.at[cid, pl.ds(nxt, NB)],
                           idx_dv.at[(p + 2) % 3], isems[1])
        if kk == 4:
          @pl.when(s <= SS - 2)
          def _():
            _prefetch()
        else:
          _prefetch()
      pltpu.async_copy(x_hbm.at[idx_sv.at[(p + 1) % 3, b]], rows[rprev],
                       gsems[rprev])
    return 0
  lax.fori_loop(0, SS, six, 0)

  for k, r in ((CPT - 2, (CPT - 2) % 3), (CPT - 1, (CPT - 1) % 3)):
    b = k % 2
    pltpu.make_async_copy(x_hbm.at[idx_sv.at[0, b]], rows[r],
                          gsems[r]).wait()
    pltpu.async_copy(rows[r], acc.at[idx_dv.at[0, b]], ssems[r], add=True)
    if compute_cnt:
      pltpu.async_copy(ones.at[pl.ds(0, CH)], cnt.at[idx_dv.at[0, b]],
                       csems[r], add=True)
    _wait_scatter((k + 2) % 3)
  _wait_scatter((CPT - 1) % 3)

  plsc.subcore_barrier()

  pltpu.sync_copy(acc.at[pl.ds(row0, ROWS_PER_TILE)],
                  out_agg.at[cid, pl.ds(row0, ROWS_PER_TILE)])
  if compute_cnt:
    pltpu.sync_copy(cnt.at[pl.ds(row0, ROWS_PER_TILE)],
                    out_cnt.at[cid, pl.ds(row0, ROWS_PER_TILE)])


def _make_segsum(compute_cnt):
  mesh = plsc.VectorSubcoreMesh(core_axis_name="c", subcore_axis_name="s",
                                num_cores=NC, num_subcores=NS)
  out_type = [jax.ShapeDtypeStruct((NC, N_PAD, D), jnp.float32)]
  scratch = [pltpu.VMEM_SHARED((N_PAD, D), jnp.float32)]
  if compute_cnt:
    out_type.append(jax.ShapeDtypeStruct((NC, N_PAD), jnp.float32))
    scratch.append(pltpu.VMEM_SHARED((N_PAD,), jnp.float32))
  scratch.append(pltpu.VMEM((32, D), jnp.float32))
  if compute_cnt:
    scratch.append(pltpu.VMEM((ROWS_PER_TILE,), jnp.float32))
    scratch.append(pltpu.VMEM((128,), jnp.float32))
  scratch += [
      pltpu.VMEM((3, NB, CH), jnp.int32),
      pltpu.VMEM((3, NB, CH), jnp.int32),
  ]
  scratch += [pltpu.VMEM((CH, D), jnp.float32)] * RB
  scratch += [pltpu.SemaphoreType.DMA] * (RB * (3 if compute_cnt else 2) + 2)
  return pl.kernel(
      functools.partial(_seg_body, compute_cnt),
      out_type=tuple(out_type),
      mesh=mesh,
      scratch_types=tuple(scratch),
  )


_segsum_cnt = _make_segsum(True)
_segsum = _make_segsum(False)


def _tc_body(last, agg_ref, cnts_ref, cntc_ref, x_ref,
             wls_ref, wlc_ref, wrs_ref, wrc_ref, bs_ref, bc_ref,
             *rest):
  if last:
    wout_ref, bout_ref, out_ref = rest
  else:
    (out_ref,) = rest
  cs = cnts_ref[0, 0]
  cc = cntc_ref[0, 0]
  means = agg_ref[0] * (1.0 / jnp.maximum(cs, 1.0))[:, None]
  meanc = agg_ref[1] * (1.0 / jnp.maximum(cc, 1.0))[:, None]
  x = x_ref[...]
  h = (jnp.dot(means, wls_ref[...], preferred_element_type=jnp.float32)
       + jnp.dot(meanc, wlc_ref[...], preferred_element_type=jnp.float32)
       + jnp.dot(x, wrs_ref[...] + wrc_ref[...],
                 preferred_element_type=jnp.float32)
       + bs_ref[...] + bc_ref[...])
  h = jnp.maximum(h, 0.0)
  if last:
    out_ref[...] = (jnp.dot(h, wout_ref[...], preferred_element_type=jnp.float32)
                    + bout_ref[...])
  else:
    out_ref[...] = h


def _make_tc(last):
  agg_spec = pl.BlockSpec((NC, BN, D), lambda i: (0, i, 0))
  cnt_spec = pl.BlockSpec((1, 1, BN), lambda i: (i, 0, 0))
  x_spec = pl.BlockSpec((BN, D), lambda i: (i, 0))
  w_spec = pl.BlockSpec((D, D), lambda i: (0, 0))
  b_spec = pl.BlockSpec((1, D), lambda i: (0, 0))
  in_specs = [agg_spec, cnt_spec, cnt_spec, x_spec,
              w_spec, w_spec, w_spec, w_spec, b_spec, b_spec]
  if last:
    in_specs += [pl.BlockSpec((D, D_OUT), lambda i: (0, 0)),
                 pl.BlockSpec((1, D_OUT), lambda i: (0, 0))]
    out_spec = pl.BlockSpec((BN, D_OUT), lambda i: (i, 0))
    out_shape = jax.ShapeDtypeStruct((N, D_OUT), jnp.float32)
  else:
    out_spec = pl.BlockSpec((BN, D), lambda i: (i, 0))
    out_shape = jax.ShapeDtypeStruct((N, D), jnp.float32)
  return pl.pallas_call(
      functools.partial(_tc_body, last),
      grid=(GRID,),
      in_specs=in_specs,
      out_specs=out_spec,
      out_shape=out_shape,
  )


_tc_layer = _make_tc(False)
_tc_last = _make_tc(True)


def kernel(x, edge_index_supplies, edge_index_competes,
           Wl0_s, bl0_s, Wr0_s, Wl0_c, bl0_c, Wr0_c,
           Wl1_s, bl1_s, Wr1_s, Wl1_c, bl1_c, Wr1_c,
           W_out, b_out):
  src_all = jnp.stack([edge_index_supplies[0].reshape(N_CHUNKS, CH),
                       edge_index_competes[0].reshape(N_CHUNKS, CH)])
  dst_all = jnp.stack([edge_index_supplies[1].reshape(N_CHUNKS, CH),
                       edge_index_competes[1].reshape(N_CHUNKS, CH)])

  agg0, cnt_all = _segsum_cnt(x, src_all, dst_all)
  cnt_s_r = cnt_all[0, :N].reshape(GRID, 1, BN)
  cnt_c_r = cnt_all[1, :N].reshape(GRID, 1, BN)

  bl0_s2 = bl0_s.reshape(1, D)
  bl0_c2 = bl0_c.reshape(1, D)
  bl1_s2 = bl1_s.reshape(1, D)
  bl1_c2 = bl1_c.reshape(1, D)
  b_out2 = b_out.reshape(1, D_OUT)

  h = _tc_layer(agg0, cnt_s_r, cnt_c_r, x,
                Wl0_s, Wl0_c, Wr0_s, Wr0_c, bl0_s2, bl0_c2)

  (agg1,) = _segsum(h, src_all, dst_all)

  out = _tc_last(agg1, cnt_s_r, cnt_c_r, h,
                 Wl1_s, Wl1_c, Wr1_s, Wr1_c, bl1_s2, bl1_c2,
                 W_out, b_out2)
  return out


# speedup vs baseline: 11.7732x; 1.0069x over previous
import functools

import jax
import jax.numpy as jnp
from jax import lax
from jax.experimental import pallas as pl
from jax.experimental.pallas import tpu as pltpu
from jax.experimental.pallas import tpu_sc as plsc

N = 10000
E = 320000
D = 128
D_OUT = 64

NC = 2
NS = 16

CH = 100
N_CHUNKS = E // CH
CPT = N_CHUNKS // NS
NB = 2
GROUPS = CPT // NB
RB = 3
SS = (CPT - 2) // 6


N_PAD = 10240
ROWS_PER_TILE = N_PAD // NS

BN = 1000
GRID = N // BN


def _seg_body(compute_cnt, x_hbm, src_hbm, dst_hbm, *rest):
  if compute_cnt:
    (out_agg, out_cnt, acc, cnt, zbuf, cbuf, ones, idx_sv, idx_dv) = rest[:9]
    rest = rest[9:]
  else:
    (out_agg, acc, zbuf, idx_sv, idx_dv) = rest[:5]
    rest = rest[5:]
  rows, rest = rest[:RB], rest[RB:]
  gsems, rest = rest[:RB], rest[RB:]
  ssems, rest = rest[:RB], rest[RB:]
  if compute_cnt:
    csems, rest = rest[:RB], rest[RB:]
  isems = rest[:2]

  cid = lax.axis_index("c")
  sid = lax.axis_index("s")

  def zloop(i, _):
    zbuf[i // 8, pl.ds((i % 8) * 16, 16)] = jnp.zeros((16,), jnp.float32)
    return 0
  lax.fori_loop(0, 32 * (D // 16), zloop, 0)
  base = sid * CPT
  pltpu.sync_copy(src_hbm.at[cid, pl.ds(base, NB)], idx_sv.at[0])
  pltpu.sync_copy(dst_hbm.at[cid, pl.ds(base, NB)], idx_dv.at[0])
  pltpu.async_copy(x_hbm.at[idx_sv.at[0, 0]], rows[0], gsems[0])
  pltpu.async_copy(x_hbm.at[idx_sv.at[0, 1]], rows[1], gsems[1])
  pltpu.sync_copy(src_hbm.at[cid, pl.ds(base + NB, NB)], idx_sv.at[1])
  pltpu.sync_copy(dst_hbm.at[cid, pl.ds(base + NB, NB)], idx_dv.at[1])

  row0 = sid * ROWS_PER_TILE
  NZ = ROWS_PER_TILE // 32
  for t in range(NZ):
    pltpu.async_copy(zbuf, acc.at[pl.ds(row0 + t * 32, 32)], ssems[0])
  if compute_cnt:
    def czloop(i, _):
      cbuf[pl.ds(i * 16, 16)] = jnp.zeros((16,), jnp.float32)
      return 0
    lax.fori_loop(0, ROWS_PER_TILE // 16, czloop, 0)
    pltpu.async_copy(cbuf, cnt.at[pl.ds(row0, ROWS_PER_TILE)], ssems[1])
    def oloop(i, _):
      ones[pl.ds(i * 16, 16)] = jnp.ones((16,), jnp.float32)
      return 0
    lax.fori_loop(0, 8, oloop, 0)
    pltpu.make_async_copy(cbuf, cnt.at[pl.ds(row0, ROWS_PER_TILE)],
                          ssems[1]).wait()
  for t in range(NZ):
    pltpu.make_async_copy(zbuf, acc.at[pl.ds(row0, 32)], ssems[0]).wait()

  plsc.subcore_barrier()

  def _wait_scatter(r):
    pltpu.make_async_copy(rows[r], acc.at[idx_dv.at[0, 0]], ssems[r]).wait()
    if compute_cnt:
      pltpu.make_async_copy(ones.at[pl.ds(0, CH)], cnt.at[idx_dv.at[0, 0]],
                            csems[r]).wait()

  def _chunk(p, b, r):
    pltpu.make_async_copy(x_hbm.at[idx_sv.at[p, b]], rows[r],
                          gsems[r]).wait()
    pltpu.async_copy(rows[r], acc.at[idx_dv.at[p, b]], ssems[r], add=True)
    if compute_cnt:
      pltpu.async_copy(ones.at[pl.ds(0, CH)], cnt.at[idx_dv.at[p, b]],
                       csems[r], add=True)

  def six(s, _):
    k0 = s * 6
    for kk in range(6):
      b = kk % 2
      p = kk // 2
      r = kk % 3
      rprev = (kk + 2) % 3
      _chunk(p, b, r)
      if kk == 0:
        @pl.when(s >= 1)
        def _():
          _wait_scatter(rprev)
      else:
        _wait_scatter(rprev)
      if b == 0:
        if kk == 0:
          @pl.when(s >= 1)
          def _():
            pltpu.make_async_copy(src_hbm.at[cid, pl.ds(base, NB)],
                                  idx_sv.at[(p + 1) % 3], isems[0]).wait()
            pltpu.make_async_copy(dst_hbm.at[cid, pl.ds(base, NB)],
                                  idx_dv.at[(p + 1) % 3], isems[1]).wait()
        else:
          pltpu.make_async_copy(src_hbm.at[cid, pl.ds(base, NB)],
                                idx_sv.at[(p + 1) % 3], isems[0]).wait()
          pltpu.make_async_copy(dst_hbm.at[cid, pl.ds(base, NB)],
                                idx_dv.at[(p + 1) % 3], isems[1]).wait()
        g = s * 3 + p
        def _prefetch(g=g, p=p):
          nxt = base + (g + 2) * NB
          pltpu.async_copy(src_hbm.at[cid, pl.ds(nxt, NB)],
                           idx_sv.at[(p + 2) % 3], isems[0])
          pltpu.async_copy(dst_hbm.at[cid, pl.ds(nxt, NB)],
                           idx_dv.at[(p + 2) % 3], isems[1])
        if kk == 4:
          @pl.when(s <= SS - 2)
          def _():
            _prefetch()
        else:
          _prefetch()
      pltpu.async_copy(x_hbm.at[idx_sv.at[(p + 1) % 3, b]], rows[rprev],
                       gsems[rprev])
    return 0
  lax.fori_loop(0, SS, six, 0)

  for k, r in ((CPT - 2, (CPT - 2) % 3), (CPT - 1, (CPT - 1) % 3)):
    b = k % 2
    pltpu.make_async_copy(x_hbm.at[idx_sv.at[0, b]], rows[r],
                          gsems[r]).wait()
    pltpu.async_copy(rows[r], acc.at[idx_dv.at[0, b]], ssems[r], add=True)
    if compute_cnt:
      pltpu.async_copy(ones.at[pl.ds(0, CH)], cnt.at[idx_dv.at[0, b]],
                       csems[r], add=True)
    _wait_scatter((k + 2) % 3)
  _wait_scatter((CPT - 1) % 3)

  plsc.subcore_barrier()

  pltpu.sync_copy(acc.at[pl.ds(row0, ROWS_PER_TILE)],
                  out_agg.at[cid, pl.ds(row0, ROWS_PER_TILE)])
  if compute_cnt:
    pltpu.sync_copy(cnt.at[pl.ds(row0, ROWS_PER_TILE)],
                    out_cnt.at[cid, pl.ds(row0, ROWS_PER_TILE)])


def _make_segsum(compute_cnt):
  mesh = plsc.VectorSubcoreMesh(core_axis_name="c", subcore_axis_name="s",
                                num_cores=NC, num_subcores=NS)
  out_type = [jax.ShapeDtypeStruct((NC, N_PAD, D), jnp.float32)]
  scratch = [pltpu.VMEM_SHARED((N_PAD, D), jnp.float32)]
  if compute_cnt:
    out_type.append(jax.ShapeDtypeStruct((NC, N_PAD), jnp.float32))
    scratch.append(pltpu.VMEM_SHARED((N_PAD,), jnp.float32))
  scratch.append(pltpu.VMEM((32, D), jnp.float32))
  if compute_cnt:
    scratch.append(pltpu.VMEM((ROWS_PER_TILE,), jnp.float32))
    scratch.append(pltpu.VMEM((128,), jnp.float32))
  scratch += [
      pltpu.VMEM((3, NB, CH), jnp.int32),
      pltpu.VMEM((3, NB, CH), jnp.int32),
  ]
  scratch += [pltpu.VMEM((CH, D), jnp.float32)] * RB
  scratch += [pltpu.SemaphoreType.DMA] * (RB * (3 if compute_cnt else 2) + 2)
  return pl.kernel(
      functools.partial(_seg_body, compute_cnt),
      out_type=tuple(out_type),
      mesh=mesh,
      scratch_types=tuple(scratch),
  )


_segsum_cnt = _make_segsum(True)
_segsum = _make_segsum(False)


def _tc_body(last, agg_ref, cnts_ref, cntc_ref, x_ref,
             wls_ref, wlc_ref, wrs_ref, wrc_ref, bs_ref, bc_ref,
             *rest):
  if last:
    wout_ref, bout_ref, out_ref = rest
  else:
    (out_ref,) = rest
  cs = cnts_ref[0, 0]
  cc = cntc_ref[0, 0]
  means = agg_ref[0] * (1.0 / jnp.maximum(cs, 1.0))[:, None]
  meanc = agg_ref[1] * (1.0 / jnp.maximum(cc, 1.0))[:, None]
  x = x_ref[...]
  h = (jnp.dot(means, wls_ref[...], preferred_element_type=jnp.float32)
       + jnp.dot(meanc, wlc_ref[...], preferred_element_type=jnp.float32)
       + jnp.dot(x, wrs_ref[...] + wrc_ref[...],
                 preferred_element_type=jnp.float32)
       + bs_ref[...] + bc_ref[...])
  h = jnp.maximum(h, 0.0)
  if last:
    out_ref[...] = (jnp.dot(h, wout_ref[...], preferred_element_type=jnp.float32)
                    + bout_ref[...])
  else:
    out_ref[...] = h


def _make_tc(last):
  agg_spec = pl.BlockSpec((NC, BN, D), lambda i: (0, i, 0))
  cnt_spec = pl.BlockSpec((1, 1, BN), lambda i: (i, 0, 0))
  x_spec = pl.BlockSpec((BN, D), lambda i: (i, 0))
  w_spec = pl.BlockSpec((D, D), lambda i: (0, 0))
  b_spec = pl.BlockSpec((1, D), lambda i: (0, 0))
  in_specs = [agg_spec, cnt_spec, cnt_spec, x_spec,
              w_spec, w_spec, w_spec, w_spec, b_spec, b_spec]
  if last:
    in_specs += [pl.BlockSpec((D, D_OUT), lambda i: (0, 0)),
                 pl.BlockSpec((1, D_OUT), lambda i: (0, 0))]
    out_spec = pl.BlockSpec((BN, D_OUT), lambda i: (i, 0))
    out_shape = jax.ShapeDtypeStruct((N, D_OUT), jnp.float32)
  else:
    out_spec = pl.BlockSpec((BN, D), lambda i: (i, 0))
    out_shape = jax.ShapeDtypeStruct((N, D), jnp.float32)
  return pl.pallas_call(
      functools.partial(_tc_body, last),
      grid=(GRID,),
      in_specs=in_specs,
      out_specs=out_spec,
      out_shape=out_shape,
  )


_tc_layer = _make_tc(False)
_tc_last = _make_tc(True)


def kernel(x, edge_index_supplies, edge_index_competes,
           Wl0_s, bl0_s, Wr0_s, Wl0_c, bl0_c, Wr0_c,
           Wl1_s, bl1_s, Wr1_s, Wl1_c, bl1_c, Wr1_c,
           W_out, b_out):
  src_all = jnp.stack([edge_index_supplies[0].reshape(N_CHUNKS, CH),
                       edge_index_competes[0].reshape(N_CHUNKS, CH)])
  dst_all = jnp.stack([edge_index_supplies[1].reshape(N_CHUNKS, CH),
                       edge_index_competes[1].reshape(N_CHUNKS, CH)])

  agg0, cnt_all = _segsum_cnt(x, src_all, dst_all)
  cnt_s_r = cnt_all[0, :N].reshape(GRID, 1, BN)
  cnt_c_r = cnt_all[1, :N].reshape(GRID, 1, BN)

  bl0_s2 = bl0_s.reshape(1, D)
  bl0_c2 = bl0_c.reshape(1, D)
  bl1_s2 = bl1_s.reshape(1, D)
  bl1_c2 = bl1_c.reshape(1, D)
  b_out2 = b_out.reshape(1, D_OUT)

  h = _tc_layer(agg0, cnt_s_r, cnt_c_r, x,
                Wl0_s, Wl0_c, Wr0_s, Wr0_c, bl0_s2, bl0_c2)

  (agg1,) = _segsum(h, src_all, dst_all)

  out = _tc_last(agg1, cnt_s_r, cnt_c_r, h,
                 Wl1_s, Wl1_c, Wr1_s, Wr1_c, bl1_s2, bl1_c2,
                 W_out, b_out2)
  return out
